# baseline (speedup 1.0000x reference)
import functools

import jax
import jax.numpy as jnp
from jax import lax
from jax.experimental import pallas as pl
from jax.experimental.pallas import tpu as pltpu
from jax.experimental.pallas import tpu_sc as plsc

B, N = 4, 16384
M, K = 1024, 32
BM = B * M
BT = BM * K
TABW = 128
MB = 256
Q = 256
P_CONV = float(BT)
EPS = 1e-5
HI = jax.lax.Precision.HIGHEST


MBL = 256
NC = 128
CL = N // NC
BIGI = 2 ** 30


def _knn_kernel(st_ref, x_ref, idx_ref, vals_ref, cm_ref, cpos_ref,
                cm2_ref, cpos2_ref, buf_ref):
    b = pl.program_id(0)
    st = st_ref[0]
    x = x_ref[0]
    s2 = jnp.sum(st * st, axis=0, keepdims=True)
    x2 = jnp.sum(x * x, axis=1, keepdims=True)
    dot = jax.lax.dot_general(x, st, (((1,), (0,)), ((), ())),
                              preferred_element_type=jnp.float32)
    d2 = (s2 + x2) - 2.0 * dot
    vals_ref[...] = jnp.reshape(d2, (NC, CL, MBL))

    p_iota3 = jax.lax.broadcasted_iota(jnp.int32, (NC, CL, MBL), 1)
    g_iota3 = (jax.lax.broadcasted_iota(jnp.int32, (NC, CL, MBL), 0) * CL
               + p_iota3)
    c_iota2 = jax.lax.broadcasted_iota(jnp.int32, (NC, MBL), 0)
    k_iota2 = jax.lax.broadcasted_iota(jnp.int32, (K, MBL), 0)
    INF = jnp.float32(jnp.inf)

    def refresh(lv, lg):
        v3 = vals_ref[...]
        done = (v3 < lv) | ((v3 == lv) & (g_iota3 <= lg))
        vm = jnp.where(done, INF, v3)
        cm = jnp.min(vm, axis=1)
        cpos = jnp.min(
            jnp.where(vm == cm[:, None, :], p_iota3, CL), axis=1)
        vm2 = jnp.where(p_iota3 == cpos[:, None, :], INF, vm)
        cm2 = jnp.min(vm2, axis=1)
        cpos2 = jnp.min(
            jnp.where(vm2 == cm2[:, None, :], p_iota3, CL), axis=1)
        cm_ref[...] = cm
        cpos_ref[...] = cpos.astype(jnp.int32)
        cm2_ref[...] = cm2
        cpos2_ref[...] = cpos2.astype(jnp.int32)

    def extract_step(_, carry):
        kq, guard, lv, lg = carry
        cm, cm2 = cm_ref[...], cm2_ref[...]
        mn = jnp.min(cm, axis=0, keepdims=True)
        gcand = jnp.where(cm == mn, c_iota2 * CL + cpos_ref[...], BIGI)
        gsel = jnp.min(gcand, axis=0, keepdims=True)
        valid = (mn < guard) & (kq < K)
        csel = jax.lax.shift_right_logical(gsel, 7)
        hit = (c_iota2 == csel) & valid
        sec = hit & (cm2 == INF)
        anysec = jnp.max(jnp.where(sec, 1, 0), axis=0, keepdims=True) == 1
        cm_ref[...] = jnp.where(hit, jnp.where(sec, INF, cm2), cm)
        cpos_ref[...] = jnp.where(hit, cpos2_ref[...], cpos_ref[...])
        cm2_ref[...] = jnp.where(hit, INF, cm2)
        buf_ref[...] = jnp.where((k_iota2 == kq) & valid, gsel, buf_ref[...])
        lv = jnp.where(valid, mn, lv)
        lg = jnp.where(valid, gsel, lg)
        guard = jnp.where(anysec, jnp.minimum(guard, mn), guard)
        kq = jnp.where(valid, kq + 1, kq)
        return kq, guard, lv, lg

    def round_body(carry):
        r, kq, lv, lg = carry
        refresh(lv, lg)
        guard = jnp.full((1, MBL), INF)
        kq, _, lv, lg = jax.lax.fori_loop(
            0, K, extract_step, (kq, guard, lv, lg))
        return r + 1, kq, lv, lg

    def round_cond(carry):
        r, kq, lv, lg = carry
        return (r < K + 1) & (jnp.min(kq) < K)

    init = (jnp.int32(0), jnp.zeros((1, MBL), jnp.int32),
            jnp.full((1, MBL), -INF), jnp.full((1, MBL), -BIGI))
    jax.lax.while_loop(round_cond, round_body, init)
    idx_ref[0, 0] = buf_ref[...] + b * N


def _knn_topk(sampled_t, xyz):
    return pl.pallas_call(
        _knn_kernel,
        grid=(B, M // MBL),
        in_specs=[
            pl.BlockSpec((1, 3, MBL), lambda b, i: (b, 0, i)),
            pl.BlockSpec((1, N, 3), lambda b, i: (b, 0, 0)),
        ],
        out_specs=pl.BlockSpec((1, 1, K, MBL), lambda b, i: (b, i, 0, 0)),
        out_shape=jax.ShapeDtypeStruct((B, M // MBL, K, MBL), jnp.int32),
        scratch_shapes=[pltpu.VMEM((NC, CL, MBL), jnp.float32),
                        pltpu.VMEM((NC, MBL), jnp.float32),
                        pltpu.VMEM((NC, MBL), jnp.int32),
                        pltpu.VMEM((NC, MBL), jnp.float32),
                        pltpu.VMEM((NC, MBL), jnp.int32),
                        pltpu.VMEM((K, MBL), jnp.int32)],
    )(sampled_t, xyz)


def _gather_rows(tab, idx_flat):
    info = plsc.get_sparse_core_info()
    nw = info.num_cores * info.num_subcores
    b_per_w = BT // nw
    chunk = 128
    n_chunks = b_per_w // chunk

    @functools.partial(
        pl.kernel,
        mesh=plsc.VectorSubcoreMesh(core_axis_name="c", subcore_axis_name="s"),
        out_type=jax.ShapeDtypeStruct((BT, TABW), jnp.float32),
        scratch_types=[
            pltpu.VMEM((chunk,), jnp.int32),
            pltpu.VMEM((chunk, TABW), jnp.float32),
            pltpu.SemaphoreType.DMA,
        ],
    )
    def gather_k(tab_hbm, idx_hbm, out_hbm, idx_v, rows_v, sem):
        wid = lax.axis_index("s") * info.num_cores + lax.axis_index("c")
        base = wid * b_per_w

        def body(i, carry):
            off = base + i * chunk
            pltpu.sync_copy(idx_hbm.at[pl.ds(off, chunk)], idx_v)
            pltpu.async_copy(tab_hbm.at[idx_v], rows_v, sem).wait()
            pltpu.sync_copy(rows_v, out_hbm.at[pl.ds(off, chunk)])
            return carry

        lax.fori_loop(0, n_chunks, body, 0)

    return gather_k(tab, idx_flat)


def _grouped_kernel(g_ref, s_ref, x0_ref, gf_ref, m0_ref, s0_ref):
    @pl.when(pl.program_id(0) == 0)
    def _():
        m0_ref[...] = jnp.zeros_like(m0_ref)
        s0_ref[...] = jnp.zeros_like(s0_ref)

    g = g_ref[...]
    s = s_ref[...]
    feat = g[:, :, 0:64]
    xyzk = g[:, :, 64:67]
    rela = xyzk - s[:, None, :]
    dist = jnp.sqrt(jnp.sum(rela * rela, axis=2, keepdims=True))
    x0 = jnp.concatenate([rela, dist, feat], axis=2)
    x0_ref[...] = x0
    x2d = jnp.reshape(x0, (Q * K, 68))
    gf_ref[0] = jnp.transpose(x2d)
    m0_ref[...] += jnp.sum(x2d, axis=0, keepdims=True)
    s0_ref[...] += jax.lax.dot_general(
        x2d, x2d, (((0,), (0,)), ((), ())), precision=HI,
        preferred_element_type=jnp.float32)


def _grouped(grows, samp):
    return pl.pallas_call(
        _grouped_kernel,
        grid=(BM // Q,),
        in_specs=[
            pl.BlockSpec((Q, K, TABW), lambda i: (i, 0, 0)),
            pl.BlockSpec((Q, 3), lambda i: (i, 0)),
        ],
        out_specs=[
            pl.BlockSpec((Q, K, 68), lambda i: (i, 0, 0)),
            pl.BlockSpec((1, 68, Q * K), lambda i: (i // (M // Q), 0,
                                                    i % (M // Q))),
            pl.BlockSpec((1, 68), lambda i: (0, 0)),
            pl.BlockSpec((68, 68), lambda i: (0, 0)),
        ],
        out_shape=[
            jax.ShapeDtypeStruct((BM, K, 68), jnp.float32),
            jax.ShapeDtypeStruct((B, 68, M * K), jnp.float32),
            jax.ShapeDtypeStruct((1, 68), jnp.float32),
            jax.ShapeDtypeStruct((68, 68), jnp.float32),
        ],
    )(grows, samp)


def _make_conv_kernel(cin, cout, with_moments):
    def body(x_ref, wt_ref, g_ref, b_ref, m_ref, s_ref, *rest):
        scale_ref, shift_ref = rest[-2:]
        if with_moments:
            y_ref, mo_ref, so_ref = rest[:-2]
        else:
            (y_ref,) = rest[:-2]

        @pl.when(pl.program_id(0) == 0)
        def _():
            wt = wt_ref[...]
            mu = jax.lax.dot_general(
                m_ref[...] / P_CONV, wt, (((1,), (0,)), ((), ())),
                precision=HI, preferred_element_type=jnp.float32)
            t = jax.lax.dot_general(
                s_ref[...] / P_CONV, wt, (((1,), (0,)), ((), ())),
                precision=HI, preferred_element_type=jnp.float32)
            e2 = jnp.sum(wt * t, axis=0, keepdims=True)
            var = e2 - mu * mu
            isd = jax.lax.rsqrt(var + EPS)
            scale_ref[...] = isd * g_ref[...]
            shift_ref[...] = b_ref[...] - mu * isd * g_ref[...]
            if with_moments:
                mo_ref[...] = jnp.zeros_like(mo_ref)
                so_ref[...] = jnp.zeros_like(so_ref)

        x = jnp.reshape(x_ref[...], (Q * K, cin))
        y = jax.lax.dot_general(x, wt_ref[...], (((1,), (0,)), ((), ())),
                                preferred_element_type=jnp.float32)
        y = jnp.maximum(y * scale_ref[...] + shift_ref[...], 0.0)
        y_ref[...] = jnp.reshape(y, (Q, K, cout))
        if with_moments:
            mo_ref[...] += jnp.sum(y, axis=0, keepdims=True)
            so_ref[...] += jax.lax.dot_general(
                y, y, (((0,), (0,)), ((), ())), precision=HI,
                preferred_element_type=jnp.float32)

    return body


def _conv_layer(x, wt, g, b, m, s, cin, cout, with_moments):
    kern = _make_conv_kernel(cin, cout, with_moments)
    out_specs = [pl.BlockSpec((Q, K, cout), lambda i: (i, 0, 0))]
    out_shape = [jax.ShapeDtypeStruct((BM, K, cout), jnp.float32)]
    if with_moments:
        out_specs += [pl.BlockSpec((1, cout), lambda i: (0, 0)),
                      pl.BlockSpec((cout, cout), lambda i: (0, 0))]
        out_shape += [jax.ShapeDtypeStruct((1, cout), jnp.float32),
                      jax.ShapeDtypeStruct((cout, cout), jnp.float32)]
    return pl.pallas_call(
        functools.partial(kern),
        grid=(BM // Q,),
        in_specs=[
            pl.BlockSpec((Q, K, cin), lambda i: (i, 0, 0)),
            pl.BlockSpec((cin, cout), lambda i: (0, 0)),
            pl.BlockSpec((1, cout), lambda i: (0, 0)),
            pl.BlockSpec((1, cout), lambda i: (0, 0)),
            pl.BlockSpec((1, cin), lambda i: (0, 0)),
            pl.BlockSpec((cin, cin), lambda i: (0, 0)),
        ],
        out_specs=out_specs,
        out_shape=out_shape,
        scratch_shapes=[pltpu.VMEM((1, cout), jnp.float32),
                        pltpu.VMEM((1, cout), jnp.float32)],
    )(x, wt, g, b, m, s)


def _attn_kernel(x_ref, wt_ref, g_ref, b_ref, m_ref, s_ref, kx_ref,
                 afm_ref, af_ref, kp_ref, scale_ref, shift_ref):
    @pl.when(pl.program_id(0) == 0)
    def _():
        wt = wt_ref[...]
        mu = jax.lax.dot_general(
            m_ref[...] / P_CONV, wt, (((1,), (0,)), ((), ())),
            precision=HI, preferred_element_type=jnp.float32)
        t = jax.lax.dot_general(
            s_ref[...] / P_CONV, wt, (((1,), (0,)), ((), ())),
            precision=HI, preferred_element_type=jnp.float32)
        e2 = jnp.sum(wt * t, axis=0, keepdims=True)
        var = e2 - mu * mu
        isd = jax.lax.rsqrt(var + EPS)
        scale_ref[...] = isd * g_ref[...]
        shift_ref[...] = b_ref[...] - mu * isd * g_ref[...]

    x = jnp.reshape(x_ref[...], (Q * K, 128))
    y = jax.lax.dot_general(x, wt_ref[...], (((1,), (0,)), ((), ())),
                            preferred_element_type=jnp.float32)
    emb = jnp.maximum(y * scale_ref[...] + shift_ref[...], 0.0)
    emb3 = jnp.reshape(emb, (Q, K, 256))
    x1max = jnp.max(emb3, axis=2)
    mx = jnp.max(x1max, axis=1, keepdims=True)
    e = jnp.exp(x1max - mx)
    aw = e / jnp.sum(e, axis=1, keepdims=True)
    kp_ref[...] = jnp.sum(aw[:, :, None] * kx_ref[...], axis=1)
    afm = emb3 * aw[:, :, None]
    afm_ref[0] = jnp.transpose(jnp.reshape(afm, (Q * K, 256)))
    af_ref[...] = jnp.sum(afm, axis=1)


def _attn(x2, w3t, g3, b3, m2, s2, knn_xyz):
    return pl.pallas_call(
        _attn_kernel,
        grid=(BM // Q,),
        in_specs=[
            pl.BlockSpec((Q, K, 128), lambda i: (i, 0, 0)),
            pl.BlockSpec((128, 256), lambda i: (0, 0)),
            pl.BlockSpec((1, 256), lambda i: (0, 0)),
            pl.BlockSpec((1, 256), lambda i: (0, 0)),
            pl.BlockSpec((1, 128), lambda i: (0, 0)),
            pl.BlockSpec((128, 128), lambda i: (0, 0)),
            pl.BlockSpec((Q, K, 3), lambda i: (i, 0, 0)),
        ],
        out_specs=[
            pl.BlockSpec((1, 256, Q * K), lambda i: (i // (M // Q), 0,
                                                     i % (M // Q))),
            pl.BlockSpec((Q, 256), lambda i: (i, 0)),
            pl.BlockSpec((Q, 3), lambda i: (i, 0)),
        ],
        out_shape=[
            jax.ShapeDtypeStruct((B, 256, M * K), jnp.float32),
            jax.ShapeDtypeStruct((BM, 256), jnp.float32),
            jax.ShapeDtypeStruct((BM, 3), jnp.float32),
        ],
        scratch_shapes=[pltpu.VMEM((1, 256), jnp.float32),
                        pltpu.VMEM((1, 256), jnp.float32)],
    )(x2, w3t, g3, b3, m2, s2, knn_xyz)


def _head_kernel(af_ref, w1t_ref, b1_ref, g1_ref, bg1_ref,
                 w2t_ref, b2_ref, g2_ref, bg2_ref, w3t_ref, b3_ref,
                 sig_ref):
    af = af_ref[...]

    def mlp_bn(x, wt, bb, g, b):
        y = jax.lax.dot_general(x, wt, (((1,), (0,)), ((), ())),
                                preferred_element_type=jnp.float32) + bb
        mu = jnp.mean(y, axis=0, keepdims=True)
        var = jnp.mean((y - mu) * (y - mu), axis=0, keepdims=True)
        return jnp.maximum((y - mu) * jax.lax.rsqrt(var + EPS) * g + b, 0.0)

    h = mlp_bn(af, w1t_ref[...], b1_ref[...], g1_ref[...], bg1_ref[...])
    h = mlp_bn(h, w2t_ref[...], b2_ref[...], g2_ref[...], bg2_ref[...])
    s = jax.lax.dot_general(h, w3t_ref[...], (((1,), (0,)), ((), ())),
                            preferred_element_type=jnp.float32) + b3_ref[...]
    sp = jnp.maximum(s, 0.0) + jnp.log(1.0 + jnp.exp(-jnp.abs(s)))
    sig_ref[...] = sp + 0.001


def _head(af, wm1t, bm1, gm1, bgm1, wm2t, bm2, gm2, bgm2, wm3t, bm3):
    return pl.pallas_call(
        _head_kernel,
        out_shape=jax.ShapeDtypeStruct((BM, 1), jnp.float32),
    )(af, wm1t, bm1, gm1, bgm1, wm2t, bm2, gm2, bgm2, wm3t, bm3)


def kernel(xyz, features, W1, g1, b1, W2, g2, b2, W3, g3, b3,
           Wm1, bm1, gm1, bgm1, Wm2, bm2, gm2, bgm2, Wm3, bm3):
    perm = jax.random.permutation(jax.random.key(42), N)[:M]
    sampled = jnp.take(xyz, perm, axis=1)
    samp_t = jnp.transpose(sampled, (0, 2, 1))

    idx4 = _knn_topk(samp_t, xyz)
    idx = idx4.transpose(0, 1, 3, 2).reshape(B, M, K)

    pad = jnp.zeros((B, N, TABW - 67), jnp.float32)
    tab = jnp.concatenate([features, xyz, pad], axis=2).reshape(B * N, TABW)
    grows = _gather_rows(tab, idx.reshape(BT))
    grows3 = grows.reshape(BM, K, TABW)
    knn_xyz = grows3[:, :, 64:67]

    samp2 = sampled.reshape(BM, 3)
    x0, gf, m0, s0 = _grouped(grows3, samp2)

    w1t, w2t, w3t = W1.T, W2.T, W3.T
    x1, m1, s1 = _conv_layer(x0, w1t, g1[None], b1[None], m0, s0,
                             68, 64, True)
    x2, m2, s2 = _conv_layer(x1, w2t, g2[None], b2[None], m1, s1,
                             64, 128, True)
    afm, af, kp = _attn(x2, w3t, g3[None], b3[None], m2, s2, knn_xyz)

    sig = _head(af, Wm1.T, bm1[None], gm1[None], bgm1[None],
                Wm2.T, bm2[None], gm2[None], bgm2[None], Wm3.T, bm3[None])

    keypoints = kp.reshape(B, M, 3)
    sigmas = sig.reshape(B, M)
    attentive_feature = af.reshape(B, M, 256).transpose(0, 2, 1)
    grouped_features = gf.reshape(B, 68, M, K)
    attentive_feature_map = afm.reshape(B, 256, M, K)
    return (keypoints, sigmas, attentive_feature, grouped_features,
            attentive_feature_map)


# speedup vs baseline: 1.0666x; 1.0666x over previous
import functools

import jax
import jax.numpy as jnp
from jax import lax
from jax.experimental import pallas as pl
from jax.experimental.pallas import tpu as pltpu
from jax.experimental.pallas import tpu_sc as plsc

B, N = 4, 16384
M, K = 1024, 32
BM = B * M
BT = BM * K
TABW = 128
MB = 256
Q = 256
P_CONV = float(BT)
EPS = 1e-5
HI = jax.lax.Precision.HIGHEST


MBL = 256
NC = 256
CL = N // NC
CSH = CL.bit_length() - 1
BIGI = 2 ** 30


def _knn_kernel(st_ref, x_ref, idx_ref, vals_ref, cm_ref, cpos_ref,
                cm2_ref, cpos2_ref, buf_ref):
    b = pl.program_id(0)
    st = st_ref[0]
    x = x_ref[0]
    s2 = jnp.sum(st * st, axis=0, keepdims=True)
    x2 = jnp.sum(x * x, axis=1, keepdims=True)
    dot = jax.lax.dot_general(x, st, (((1,), (0,)), ((), ())),
                              preferred_element_type=jnp.float32)
    d2 = (s2 + x2) - 2.0 * dot
    vals_ref[...] = jnp.reshape(d2, (NC, CL, MBL))

    p_iota3 = jax.lax.broadcasted_iota(jnp.int32, (NC, CL, MBL), 1)
    g_iota3 = (jax.lax.broadcasted_iota(jnp.int32, (NC, CL, MBL), 0) * CL
               + p_iota3)
    c_iota2 = jax.lax.broadcasted_iota(jnp.int32, (NC, MBL), 0)
    k_iota2 = jax.lax.broadcasted_iota(jnp.int32, (K, MBL), 0)
    INF = jnp.float32(jnp.inf)

    def refresh(lv, lg):
        v3 = vals_ref[...]
        done = (v3 < lv) | ((v3 == lv) & (g_iota3 <= lg))
        vm = jnp.where(done, INF, v3)
        cm = jnp.min(vm, axis=1)
        cpos = jnp.min(
            jnp.where(vm == cm[:, None, :], p_iota3, CL), axis=1)
        vm2 = jnp.where(p_iota3 == cpos[:, None, :], INF, vm)
        cm2 = jnp.min(vm2, axis=1)
        cpos2 = jnp.min(
            jnp.where(vm2 == cm2[:, None, :], p_iota3, CL), axis=1)
        cm_ref[...] = cm
        cpos_ref[...] = cpos.astype(jnp.int32)
        cm2_ref[...] = cm2
        cpos2_ref[...] = cpos2.astype(jnp.int32)

    def extract_step(_, carry):
        kq, guard, lv, lg = carry
        cm, cm2 = cm_ref[...], cm2_ref[...]
        mn = jnp.min(cm, axis=0, keepdims=True)
        gcand = jnp.where(cm == mn, c_iota2 * CL + cpos_ref[...], BIGI)
        gsel = jnp.min(gcand, axis=0, keepdims=True)
        valid = (mn < guard) & (kq < K)
        csel = jax.lax.shift_right_logical(gsel, CSH)
        hit = (c_iota2 == csel) & valid
        sec = hit & (cm2 == INF)
        anysec = jnp.max(jnp.where(sec, 1, 0), axis=0, keepdims=True) == 1
        cm_ref[...] = jnp.where(hit, jnp.where(sec, INF, cm2), cm)
        cpos_ref[...] = jnp.where(hit, cpos2_ref[...], cpos_ref[...])
        cm2_ref[...] = jnp.where(hit, INF, cm2)
        buf_ref[...] = jnp.where((k_iota2 == kq) & valid, gsel, buf_ref[...])
        lv = jnp.where(valid, mn, lv)
        lg = jnp.where(valid, gsel, lg)
        guard = jnp.where(anysec, jnp.minimum(guard, mn), guard)
        kq = jnp.where(valid, kq + 1, kq)
        return kq, guard, lv, lg

    def round_body(carry):
        r, kq, lv, lg = carry
        refresh(lv, lg)
        guard = jnp.full((1, MBL), INF)
        kq, _, lv, lg = jax.lax.fori_loop(
            0, K, extract_step, (kq, guard, lv, lg))
        return r + 1, kq, lv, lg

    def round_cond(carry):
        r, kq, lv, lg = carry
        return (r < K + 1) & (jnp.min(kq) < K)

    init = (jnp.int32(0), jnp.zeros((1, MBL), jnp.int32),
            jnp.full((1, MBL), -INF), jnp.full((1, MBL), -BIGI))
    jax.lax.while_loop(round_cond, round_body, init)
    idx_ref[0, 0] = buf_ref[...] + b * N


def _knn_topk(sampled_t, xyz):
    return pl.pallas_call(
        _knn_kernel,
        grid=(B, M // MBL),
        in_specs=[
            pl.BlockSpec((1, 3, MBL), lambda b, i: (b, 0, i)),
            pl.BlockSpec((1, N, 3), lambda b, i: (b, 0, 0)),
        ],
        out_specs=pl.BlockSpec((1, 1, K, MBL), lambda b, i: (b, i, 0, 0)),
        out_shape=jax.ShapeDtypeStruct((B, M // MBL, K, MBL), jnp.int32),
        scratch_shapes=[pltpu.VMEM((NC, CL, MBL), jnp.float32),
                        pltpu.VMEM((NC, MBL), jnp.float32),
                        pltpu.VMEM((NC, MBL), jnp.int32),
                        pltpu.VMEM((NC, MBL), jnp.float32),
                        pltpu.VMEM((NC, MBL), jnp.int32),
                        pltpu.VMEM((K, MBL), jnp.int32)],
        compiler_params=pltpu.CompilerParams(
            vmem_limit_bytes=100 * 1024 * 1024),
    )(sampled_t, xyz)


def _gather_rows(tab, idx_flat):
    info = plsc.get_sparse_core_info()
    nw = info.num_cores * info.num_subcores
    b_per_w = BT // nw
    chunk = 128
    n_chunks = b_per_w // chunk

    @functools.partial(
        pl.kernel,
        mesh=plsc.VectorSubcoreMesh(core_axis_name="c", subcore_axis_name="s"),
        out_type=jax.ShapeDtypeStruct((BT, TABW), jnp.float32),
        scratch_types=[
            pltpu.VMEM((chunk,), jnp.int32),
            pltpu.VMEM((chunk, TABW), jnp.float32),
            pltpu.SemaphoreType.DMA,
        ],
    )
    def gather_k(tab_hbm, idx_hbm, out_hbm, idx_v, rows_v, sem):
        wid = lax.axis_index("s") * info.num_cores + lax.axis_index("c")
        base = wid * b_per_w

        def body(i, carry):
            off = base + i * chunk
            pltpu.sync_copy(idx_hbm.at[pl.ds(off, chunk)], idx_v)
            pltpu.async_copy(tab_hbm.at[idx_v], rows_v, sem).wait()
            pltpu.sync_copy(rows_v, out_hbm.at[pl.ds(off, chunk)])
            return carry

        lax.fori_loop(0, n_chunks, body, 0)

    return gather_k(tab, idx_flat)


def _grouped_kernel(g_ref, s_ref, x0_ref, m0_ref, s0_ref):
    @pl.when(pl.program_id(0) == 0)
    def _():
        m0_ref[...] = jnp.zeros_like(m0_ref)
        s0_ref[...] = jnp.zeros_like(s0_ref)

    g = g_ref[...]
    s = s_ref[...]
    feat = g[:, :, 0:64]
    xyzk = g[:, :, 64:67]
    rela = xyzk - s[:, None, :]
    dist = jnp.sqrt(jnp.sum(rela * rela, axis=2, keepdims=True))
    x0 = jnp.concatenate([rela, dist, feat], axis=2)
    x0_ref[...] = x0
    x2d = jnp.reshape(x0, (Q * K, 68))
    m0_ref[...] += jnp.sum(x2d, axis=0, keepdims=True)
    s0_ref[...] += jax.lax.dot_general(
        x2d, x2d, (((0,), (0,)), ((), ())), precision=HI,
        preferred_element_type=jnp.float32)


def _grouped(grows, samp):
    return pl.pallas_call(
        _grouped_kernel,
        grid=(BM // Q,),
        in_specs=[
            pl.BlockSpec((Q, K, TABW), lambda i: (i, 0, 0)),
            pl.BlockSpec((Q, 3), lambda i: (i, 0)),
        ],
        out_specs=[
            pl.BlockSpec((Q, K, 68), lambda i: (i, 0, 0)),
            pl.BlockSpec((1, 68), lambda i: (0, 0)),
            pl.BlockSpec((68, 68), lambda i: (0, 0)),
        ],
        out_shape=[
            jax.ShapeDtypeStruct((BM, K, 68), jnp.float32),
            jax.ShapeDtypeStruct((1, 68), jnp.float32),
            jax.ShapeDtypeStruct((68, 68), jnp.float32),
        ],
    )(grows, samp)


def _make_conv_kernel(cin, cout, with_moments):
    def body(x_ref, wt_ref, g_ref, b_ref, m_ref, s_ref, *rest):
        scale_ref, shift_ref = rest[-2:]
        if with_moments:
            y_ref, mo_ref, so_ref = rest[:-2]
        else:
            (y_ref,) = rest[:-2]

        @pl.when(pl.program_id(0) == 0)
        def _():
            wt = wt_ref[...]
            mu = jax.lax.dot_general(
                m_ref[...] / P_CONV, wt, (((1,), (0,)), ((), ())),
                precision=HI, preferred_element_type=jnp.float32)
            t = jax.lax.dot_general(
                s_ref[...] / P_CONV, wt, (((1,), (0,)), ((), ())),
                precision=HI, preferred_element_type=jnp.float32)
            e2 = jnp.sum(wt * t, axis=0, keepdims=True)
            var = e2 - mu * mu
            isd = jax.lax.rsqrt(var + EPS)
            scale_ref[...] = isd * g_ref[...]
            shift_ref[...] = b_ref[...] - mu * isd * g_ref[...]
            if with_moments:
                mo_ref[...] = jnp.zeros_like(mo_ref)
                so_ref[...] = jnp.zeros_like(so_ref)

        x = jnp.reshape(x_ref[...], (Q * K, cin))
        y = jax.lax.dot_general(x, wt_ref[...], (((1,), (0,)), ((), ())),
                                preferred_element_type=jnp.float32)
        y = jnp.maximum(y * scale_ref[...] + shift_ref[...], 0.0)
        y_ref[...] = jnp.reshape(y, (Q, K, cout))
        if with_moments:
            mo_ref[...] += jnp.sum(y, axis=0, keepdims=True)
            so_ref[...] += jax.lax.dot_general(
                y, y, (((0,), (0,)), ((), ())), precision=HI,
                preferred_element_type=jnp.float32)

    return body


def _conv_layer(x, wt, g, b, m, s, cin, cout, with_moments):
    kern = _make_conv_kernel(cin, cout, with_moments)
    out_specs = [pl.BlockSpec((Q, K, cout), lambda i: (i, 0, 0))]
    out_shape = [jax.ShapeDtypeStruct((BM, K, cout), jnp.float32)]
    if with_moments:
        out_specs += [pl.BlockSpec((1, cout), lambda i: (0, 0)),
                      pl.BlockSpec((cout, cout), lambda i: (0, 0))]
        out_shape += [jax.ShapeDtypeStruct((1, cout), jnp.float32),
                      jax.ShapeDtypeStruct((cout, cout), jnp.float32)]
    return pl.pallas_call(
        functools.partial(kern),
        grid=(BM // Q,),
        in_specs=[
            pl.BlockSpec((Q, K, cin), lambda i: (i, 0, 0)),
            pl.BlockSpec((cin, cout), lambda i: (0, 0)),
            pl.BlockSpec((1, cout), lambda i: (0, 0)),
            pl.BlockSpec((1, cout), lambda i: (0, 0)),
            pl.BlockSpec((1, cin), lambda i: (0, 0)),
            pl.BlockSpec((cin, cin), lambda i: (0, 0)),
        ],
        out_specs=out_specs,
        out_shape=out_shape,
        scratch_shapes=[pltpu.VMEM((1, cout), jnp.float32),
                        pltpu.VMEM((1, cout), jnp.float32)],
    )(x, wt, g, b, m, s)


def _attn_kernel(x_ref, wt_ref, g_ref, b_ref, m_ref, s_ref, kx_ref,
                 afm_ref, af_ref, kp_ref, scale_ref, shift_ref):
    @pl.when(pl.program_id(0) == 0)
    def _():
        wt = wt_ref[...]
        mu = jax.lax.dot_general(
            m_ref[...] / P_CONV, wt, (((1,), (0,)), ((), ())),
            precision=HI, preferred_element_type=jnp.float32)
        t = jax.lax.dot_general(
            s_ref[...] / P_CONV, wt, (((1,), (0,)), ((), ())),
            precision=HI, preferred_element_type=jnp.float32)
        e2 = jnp.sum(wt * t, axis=0, keepdims=True)
        var = e2 - mu * mu
        isd = jax.lax.rsqrt(var + EPS)
        scale_ref[...] = isd * g_ref[...]
        shift_ref[...] = b_ref[...] - mu * isd * g_ref[...]

    x = jnp.reshape(x_ref[...], (Q * K, 128))
    y = jax.lax.dot_general(x, wt_ref[...], (((1,), (0,)), ((), ())),
                            preferred_element_type=jnp.float32)
    emb = jnp.maximum(y * scale_ref[...] + shift_ref[...], 0.0)
    emb3 = jnp.reshape(emb, (Q, K, 256))
    x1max = jnp.max(emb3, axis=2)
    mx = jnp.max(x1max, axis=1, keepdims=True)
    e = jnp.exp(x1max - mx)
    aw = e / jnp.sum(e, axis=1, keepdims=True)
    kp_ref[...] = jnp.sum(aw[:, :, None] * kx_ref[...], axis=1)
    afm = emb3 * aw[:, :, None]
    afm_ref[...] = afm
    af_ref[...] = jnp.sum(afm, axis=1)


def _attn(x2, w3t, g3, b3, m2, s2, knn_xyz):
    return pl.pallas_call(
        _attn_kernel,
        grid=(BM // Q,),
        in_specs=[
            pl.BlockSpec((Q, K, 128), lambda i: (i, 0, 0)),
            pl.BlockSpec((128, 256), lambda i: (0, 0)),
            pl.BlockSpec((1, 256), lambda i: (0, 0)),
            pl.BlockSpec((1, 256), lambda i: (0, 0)),
            pl.BlockSpec((1, 128), lambda i: (0, 0)),
            pl.BlockSpec((128, 128), lambda i: (0, 0)),
            pl.BlockSpec((Q, K, 3), lambda i: (i, 0, 0)),
        ],
        out_specs=[
            pl.BlockSpec((Q, K, 256), lambda i: (i, 0, 0)),
            pl.BlockSpec((Q, 256), lambda i: (i, 0)),
            pl.BlockSpec((Q, 3), lambda i: (i, 0)),
        ],
        out_shape=[
            jax.ShapeDtypeStruct((BM, K, 256), jnp.float32),
            jax.ShapeDtypeStruct((BM, 256), jnp.float32),
            jax.ShapeDtypeStruct((BM, 3), jnp.float32),
        ],
        scratch_shapes=[pltpu.VMEM((1, 256), jnp.float32),
                        pltpu.VMEM((1, 256), jnp.float32)],
    )(x2, w3t, g3, b3, m2, s2, knn_xyz)


def _head_kernel(af_ref, w1t_ref, b1_ref, g1_ref, bg1_ref,
                 w2t_ref, b2_ref, g2_ref, bg2_ref, w3t_ref, b3_ref,
                 sig_ref):
    af = af_ref[...]

    def mlp_bn(x, wt, bb, g, b):
        y = jax.lax.dot_general(x, wt, (((1,), (0,)), ((), ())),
                                preferred_element_type=jnp.float32) + bb
        mu = jnp.mean(y, axis=0, keepdims=True)
        var = jnp.mean((y - mu) * (y - mu), axis=0, keepdims=True)
        return jnp.maximum((y - mu) * jax.lax.rsqrt(var + EPS) * g + b, 0.0)

    h = mlp_bn(af, w1t_ref[...], b1_ref[...], g1_ref[...], bg1_ref[...])
    h = mlp_bn(h, w2t_ref[...], b2_ref[...], g2_ref[...], bg2_ref[...])
    s = jax.lax.dot_general(h, w3t_ref[...], (((1,), (0,)), ((), ())),
                            preferred_element_type=jnp.float32) + b3_ref[...]
    sp = jnp.maximum(s, 0.0) + jnp.log(1.0 + jnp.exp(-jnp.abs(s)))
    sig_ref[...] = sp + 0.001


def _head(af, wm1t, bm1, gm1, bgm1, wm2t, bm2, gm2, bgm2, wm3t, bm3):
    return pl.pallas_call(
        _head_kernel,
        out_shape=jax.ShapeDtypeStruct((BM, 1), jnp.float32),
    )(af, wm1t, bm1, gm1, bgm1, wm2t, bm2, gm2, bgm2, wm3t, bm3)


def kernel(xyz, features, W1, g1, b1, W2, g2, b2, W3, g3, b3,
           Wm1, bm1, gm1, bgm1, Wm2, bm2, gm2, bgm2, Wm3, bm3):
    perm = jax.random.permutation(jax.random.key(42), N)[:M]
    sampled = jnp.take(xyz, perm, axis=1)
    samp_t = jnp.transpose(sampled, (0, 2, 1))

    idx4 = _knn_topk(samp_t, xyz)
    idx = idx4.transpose(0, 1, 3, 2).reshape(B, M, K)

    pad = jnp.zeros((B, N, TABW - 67), jnp.float32)
    tab = jnp.concatenate([features, xyz, pad], axis=2).reshape(B * N, TABW)
    grows = _gather_rows(tab, idx.reshape(BT))
    grows3 = grows.reshape(BM, K, TABW)
    knn_xyz = grows3[:, :, 64:67]

    samp2 = sampled.reshape(BM, 3)
    x0, m0, s0 = _grouped(grows3, samp2)

    w1t, w2t, w3t = W1.T, W2.T, W3.T
    x1, m1, s1 = _conv_layer(x0, w1t, g1[None], b1[None], m0, s0,
                             68, 64, True)
    x2, m2, s2 = _conv_layer(x1, w2t, g2[None], b2[None], m1, s1,
                             64, 128, True)
    afm, af, kp = _attn(x2, w3t, g3[None], b3[None], m2, s2, knn_xyz)

    sig = _head(af, Wm1.T, bm1[None], gm1[None], bgm1[None],
                Wm2.T, bm2[None], gm2[None], bgm2[None], Wm3.T, bm3[None])

    keypoints = kp.reshape(B, M, 3)
    sigmas = sig.reshape(B, M)
    attentive_feature = af.reshape(B, M, 256).transpose(0, 2, 1)
    grouped_features = x0.reshape(B, M, K, 68).transpose(0, 3, 1, 2)
    attentive_feature_map = afm.reshape(B, M, K, 256).transpose(0, 3, 1, 2)
    return (keypoints, sigmas, attentive_feature, grouped_features,
            attentive_feature_map)


# speedup vs baseline: 1.0879x; 1.0200x over previous
import functools

import jax
import jax.numpy as jnp
from jax import lax
from jax.experimental import pallas as pl
from jax.experimental.pallas import tpu as pltpu
from jax.experimental.pallas import tpu_sc as plsc

B, N = 4, 16384
M, K = 1024, 32
BM = B * M
BT = BM * K
TABW = 128
MB = 256
Q = 256
P_CONV = float(BT)
EPS = 1e-5
HI = jax.lax.Precision.HIGHEST


MBL = 256
NC = 128
CL = N // NC
CSH = CL.bit_length() - 1
BIGI = 2 ** 30


def _knn_kernel(st_ref, x_ref, idx_ref, vals_ref, cm_ref, cpos_ref,
                cm2_ref, cpos2_ref, buf_ref):
    b = pl.program_id(0)
    st = st_ref[0]
    x = x_ref[0]
    s2 = jnp.sum(st * st, axis=0, keepdims=True)
    x2 = jnp.sum(x * x, axis=1, keepdims=True)
    dot = jax.lax.dot_general(x, st, (((1,), (0,)), ((), ())),
                              preferred_element_type=jnp.float32)
    d2 = (s2 + x2) - 2.0 * dot
    vals_ref[...] = jnp.reshape(d2, (NC, CL, MBL))

    p_iota3 = jax.lax.broadcasted_iota(jnp.int32, (NC, CL, MBL), 1)
    g_iota3 = (jax.lax.broadcasted_iota(jnp.int32, (NC, CL, MBL), 0) * CL
               + p_iota3)
    c_iota2 = jax.lax.broadcasted_iota(jnp.int32, (NC, MBL), 0)
    k_iota2 = jax.lax.broadcasted_iota(jnp.int32, (K, MBL), 0)
    INF = jnp.float32(jnp.inf)

    def refresh(lv, lg):
        v3 = vals_ref[...]
        done = (v3 < lv) | ((v3 == lv) & (g_iota3 <= lg))
        vm = jnp.where(done, INF, v3)
        cm = jnp.min(vm, axis=1)
        cpos = jnp.min(
            jnp.where(vm == cm[:, None, :], p_iota3, CL), axis=1)
        vm2 = jnp.where(p_iota3 == cpos[:, None, :], INF, vm)
        cm2 = jnp.min(vm2, axis=1)
        cpos2 = jnp.min(
            jnp.where(vm2 == cm2[:, None, :], p_iota3, CL), axis=1)
        cm_ref[...] = cm
        cpos_ref[...] = cpos.astype(jnp.int32)
        cm2_ref[...] = cm2
        cpos2_ref[...] = cpos2.astype(jnp.int32)

    def extract_step(_, carry):
        kq, guard, lv, lg = carry
        cm, cm2 = cm_ref[...], cm2_ref[...]
        mn = jnp.min(cm, axis=0, keepdims=True)
        gcand = jnp.where(cm == mn, c_iota2 * CL + cpos_ref[...], BIGI)
        gsel = jnp.min(gcand, axis=0, keepdims=True)
        valid = (mn < guard) & (kq < K)
        csel = jax.lax.shift_right_logical(gsel, CSH)
        hit = (c_iota2 == csel) & valid
        sec = hit & (cm2 == INF)
        anysec = jnp.max(jnp.where(sec, 1, 0), axis=0, keepdims=True) == 1
        cm_ref[...] = jnp.where(hit, jnp.where(sec, INF, cm2), cm)
        cpos_ref[...] = jnp.where(hit, cpos2_ref[...], cpos_ref[...])
        cm2_ref[...] = jnp.where(hit, INF, cm2)
        buf_ref[...] = jnp.where((k_iota2 == kq) & valid, gsel, buf_ref[...])
        lv = jnp.where(valid, mn, lv)
        lg = jnp.where(valid, gsel, lg)
        guard = jnp.where(anysec, jnp.minimum(guard, mn), guard)
        kq = jnp.where(valid, kq + 1, kq)
        return kq, guard, lv, lg

    def round_body(carry):
        r, kq, lv, lg = carry
        refresh(lv, lg)
        guard = jnp.full((1, MBL), INF)
        kq, _, lv, lg = jax.lax.fori_loop(
            0, K, extract_step, (kq, guard, lv, lg))
        return r + 1, kq, lv, lg

    def round_cond(carry):
        r, kq, lv, lg = carry
        return (r < K + 1) & (jnp.min(kq) < K)

    init = (jnp.int32(0), jnp.zeros((1, MBL), jnp.int32),
            jnp.full((1, MBL), -INF), jnp.full((1, MBL), -BIGI))
    jax.lax.while_loop(round_cond, round_body, init)
    idx_ref[0, 0] = buf_ref[...] + b * N


def _knn_topk(sampled_t, xyz):
    return pl.pallas_call(
        _knn_kernel,
        grid=(B, M // MBL),
        in_specs=[
            pl.BlockSpec((1, 3, MBL), lambda b, i: (b, 0, i)),
            pl.BlockSpec((1, N, 3), lambda b, i: (b, 0, 0)),
        ],
        out_specs=pl.BlockSpec((1, 1, K, MBL), lambda b, i: (b, i, 0, 0)),
        out_shape=jax.ShapeDtypeStruct((B, M // MBL, K, MBL), jnp.int32),
        scratch_shapes=[pltpu.VMEM((NC, CL, MBL), jnp.float32),
                        pltpu.VMEM((NC, MBL), jnp.float32),
                        pltpu.VMEM((NC, MBL), jnp.int32),
                        pltpu.VMEM((NC, MBL), jnp.float32),
                        pltpu.VMEM((NC, MBL), jnp.int32),
                        pltpu.VMEM((K, MBL), jnp.int32)],
        compiler_params=pltpu.CompilerParams(
            vmem_limit_bytes=100 * 1024 * 1024),
    )(sampled_t, xyz)


def _gather_rows(tab, idx_flat):
    info = plsc.get_sparse_core_info()
    nw = info.num_cores * info.num_subcores
    b_per_w = BT // nw
    chunk = 128
    n_chunks = b_per_w // chunk

    @functools.partial(
        pl.kernel,
        mesh=plsc.VectorSubcoreMesh(core_axis_name="c", subcore_axis_name="s"),
        out_type=jax.ShapeDtypeStruct((BT, TABW), jnp.float32),
        scratch_types=[
            pltpu.VMEM((chunk,), jnp.int32),
            pltpu.VMEM((chunk, TABW), jnp.float32),
            pltpu.SemaphoreType.DMA,
        ],
    )
    def gather_k(tab_hbm, idx_hbm, out_hbm, idx_v, rows_v, sem):
        wid = lax.axis_index("s") * info.num_cores + lax.axis_index("c")
        base = wid * b_per_w

        def body(i, carry):
            off = base + i * chunk
            pltpu.sync_copy(idx_hbm.at[pl.ds(off, chunk)], idx_v)
            pltpu.async_copy(tab_hbm.at[idx_v], rows_v, sem).wait()
            pltpu.sync_copy(rows_v, out_hbm.at[pl.ds(off, chunk)])
            return carry

        lax.fori_loop(0, n_chunks, body, 0)

    return gather_k(tab, idx_flat)


def _grouped_kernel(g_ref, s_ref, x0_ref, m0_ref, s0_ref):
    @pl.when(pl.program_id(0) == 0)
    def _():
        m0_ref[...] = jnp.zeros_like(m0_ref)
        s0_ref[...] = jnp.zeros_like(s0_ref)

    g = g_ref[...]
    s = s_ref[...]
    feat = g[:, :, 0:64]
    xyzk = g[:, :, 64:67]
    rela = xyzk - s[:, None, :]
    dist = jnp.sqrt(jnp.sum(rela * rela, axis=2, keepdims=True))
    x0 = jnp.concatenate([rela, dist, feat], axis=2)
    x0_ref[...] = x0
    x2d = jnp.reshape(x0, (Q * K, 68))
    m0_ref[...] += jnp.sum(x2d, axis=0, keepdims=True)
    s0_ref[...] += jax.lax.dot_general(
        x2d, x2d, (((0,), (0,)), ((), ())), precision=HI,
        preferred_element_type=jnp.float32)


def _grouped(grows, samp):
    return pl.pallas_call(
        _grouped_kernel,
        grid=(BM // Q,),
        in_specs=[
            pl.BlockSpec((Q, K, TABW), lambda i: (i, 0, 0)),
            pl.BlockSpec((Q, 3), lambda i: (i, 0)),
        ],
        out_specs=[
            pl.BlockSpec((Q, K, 68), lambda i: (i, 0, 0)),
            pl.BlockSpec((1, 68), lambda i: (0, 0)),
            pl.BlockSpec((68, 68), lambda i: (0, 0)),
        ],
        out_shape=[
            jax.ShapeDtypeStruct((BM, K, 68), jnp.float32),
            jax.ShapeDtypeStruct((1, 68), jnp.float32),
            jax.ShapeDtypeStruct((68, 68), jnp.float32),
        ],
    )(grows, samp)


def _make_conv_kernel(cin, cout, with_moments):
    def body(x_ref, wt_ref, g_ref, b_ref, m_ref, s_ref, *rest):
        scale_ref, shift_ref = rest[-2:]
        if with_moments:
            y_ref, mo_ref, so_ref = rest[:-2]
        else:
            (y_ref,) = rest[:-2]

        @pl.when(pl.program_id(0) == 0)
        def _():
            wt = wt_ref[...]
            mu = jax.lax.dot_general(
                m_ref[...] / P_CONV, wt, (((1,), (0,)), ((), ())),
                precision=HI, preferred_element_type=jnp.float32)
            t = jax.lax.dot_general(
                s_ref[...] / P_CONV, wt, (((1,), (0,)), ((), ())),
                precision=HI, preferred_element_type=jnp.float32)
            e2 = jnp.sum(wt * t, axis=0, keepdims=True)
            var = e2 - mu * mu
            isd = jax.lax.rsqrt(var + EPS)
            scale_ref[...] = isd * g_ref[...]
            shift_ref[...] = b_ref[...] - mu * isd * g_ref[...]
            if with_moments:
                mo_ref[...] = jnp.zeros_like(mo_ref)
                so_ref[...] = jnp.zeros_like(so_ref)

        x = jnp.reshape(x_ref[...], (Q * K, cin))
        y = jax.lax.dot_general(x, wt_ref[...], (((1,), (0,)), ((), ())),
                                preferred_element_type=jnp.float32)
        y = jnp.maximum(y * scale_ref[...] + shift_ref[...], 0.0)
        y_ref[...] = jnp.reshape(y, (Q, K, cout))
        if with_moments:
            mo_ref[...] += jnp.sum(y, axis=0, keepdims=True)
            so_ref[...] += jax.lax.dot_general(
                y, y, (((0,), (0,)), ((), ())), precision=HI,
                preferred_element_type=jnp.float32)

    return body


def _conv_layer(x, wt, g, b, m, s, cin, cout, with_moments):
    kern = _make_conv_kernel(cin, cout, with_moments)
    out_specs = [pl.BlockSpec((Q, K, cout), lambda i: (i, 0, 0))]
    out_shape = [jax.ShapeDtypeStruct((BM, K, cout), jnp.float32)]
    if with_moments:
        out_specs += [pl.BlockSpec((1, cout), lambda i: (0, 0)),
                      pl.BlockSpec((cout, cout), lambda i: (0, 0))]
        out_shape += [jax.ShapeDtypeStruct((1, cout), jnp.float32),
                      jax.ShapeDtypeStruct((cout, cout), jnp.float32)]
    return pl.pallas_call(
        functools.partial(kern),
        grid=(BM // Q,),
        in_specs=[
            pl.BlockSpec((Q, K, cin), lambda i: (i, 0, 0)),
            pl.BlockSpec((cin, cout), lambda i: (0, 0)),
            pl.BlockSpec((1, cout), lambda i: (0, 0)),
            pl.BlockSpec((1, cout), lambda i: (0, 0)),
            pl.BlockSpec((1, cin), lambda i: (0, 0)),
            pl.BlockSpec((cin, cin), lambda i: (0, 0)),
        ],
        out_specs=out_specs,
        out_shape=out_shape,
        scratch_shapes=[pltpu.VMEM((1, cout), jnp.float32),
                        pltpu.VMEM((1, cout), jnp.float32)],
    )(x, wt, g, b, m, s)


def _attn_kernel(x_ref, wt_ref, g_ref, b_ref, m_ref, s_ref, kx_ref,
                 afm_ref, af_ref, kp_ref, scale_ref, shift_ref):
    @pl.when(pl.program_id(0) == 0)
    def _():
        wt = wt_ref[...]
        mu = jax.lax.dot_general(
            m_ref[...] / P_CONV, wt, (((1,), (0,)), ((), ())),
            precision=HI, preferred_element_type=jnp.float32)
        t = jax.lax.dot_general(
            s_ref[...] / P_CONV, wt, (((1,), (0,)), ((), ())),
            precision=HI, preferred_element_type=jnp.float32)
        e2 = jnp.sum(wt * t, axis=0, keepdims=True)
        var = e2 - mu * mu
        isd = jax.lax.rsqrt(var + EPS)
        scale_ref[...] = isd * g_ref[...]
        shift_ref[...] = b_ref[...] - mu * isd * g_ref[...]

    x = jnp.reshape(x_ref[...], (Q * K, 128))
    y = jax.lax.dot_general(x, wt_ref[...], (((1,), (0,)), ((), ())),
                            preferred_element_type=jnp.float32)
    emb = jnp.maximum(y * scale_ref[...] + shift_ref[...], 0.0)
    emb3 = jnp.reshape(emb, (Q, K, 256))
    x1max = jnp.max(emb3, axis=2)
    mx = jnp.max(x1max, axis=1, keepdims=True)
    e = jnp.exp(x1max - mx)
    aw = e / jnp.sum(e, axis=1, keepdims=True)
    kp_ref[...] = jnp.sum(aw[:, :, None] * kx_ref[...], axis=1)
    afm = emb3 * aw[:, :, None]
    afm_ref[...] = afm
    af_ref[...] = jnp.sum(afm, axis=1)


def _attn(x2, w3t, g3, b3, m2, s2, knn_xyz):
    return pl.pallas_call(
        _attn_kernel,
        grid=(BM // Q,),
        in_specs=[
            pl.BlockSpec((Q, K, 128), lambda i: (i, 0, 0)),
            pl.BlockSpec((128, 256), lambda i: (0, 0)),
            pl.BlockSpec((1, 256), lambda i: (0, 0)),
            pl.BlockSpec((1, 256), lambda i: (0, 0)),
            pl.BlockSpec((1, 128), lambda i: (0, 0)),
            pl.BlockSpec((128, 128), lambda i: (0, 0)),
            pl.BlockSpec((Q, K, 3), lambda i: (i, 0, 0)),
        ],
        out_specs=[
            pl.BlockSpec((Q, K, 256), lambda i: (i, 0, 0)),
            pl.BlockSpec((Q, 256), lambda i: (i, 0)),
            pl.BlockSpec((Q, 3), lambda i: (i, 0)),
        ],
        out_shape=[
            jax.ShapeDtypeStruct((BM, K, 256), jnp.float32),
            jax.ShapeDtypeStruct((BM, 256), jnp.float32),
            jax.ShapeDtypeStruct((BM, 3), jnp.float32),
        ],
        scratch_shapes=[pltpu.VMEM((1, 256), jnp.float32),
                        pltpu.VMEM((1, 256), jnp.float32)],
    )(x2, w3t, g3, b3, m2, s2, knn_xyz)


def _head_kernel(af_ref, w1t_ref, b1_ref, g1_ref, bg1_ref,
                 w2t_ref, b2_ref, g2_ref, bg2_ref, w3t_ref, b3_ref,
                 sig_ref):
    af = af_ref[...]

    def mlp_bn(x, wt, bb, g, b):
        y = jax.lax.dot_general(x, wt, (((1,), (0,)), ((), ())),
                                preferred_element_type=jnp.float32) + bb
        mu = jnp.mean(y, axis=0, keepdims=True)
        var = jnp.mean((y - mu) * (y - mu), axis=0, keepdims=True)
        return jnp.maximum((y - mu) * jax.lax.rsqrt(var + EPS) * g + b, 0.0)

    h = mlp_bn(af, w1t_ref[...], b1_ref[...], g1_ref[...], bg1_ref[...])
    h = mlp_bn(h, w2t_ref[...], b2_ref[...], g2_ref[...], bg2_ref[...])
    s = jax.lax.dot_general(h, w3t_ref[...], (((1,), (0,)), ((), ())),
                            preferred_element_type=jnp.float32) + b3_ref[...]
    sp = jnp.maximum(s, 0.0) + jnp.log(1.0 + jnp.exp(-jnp.abs(s)))
    sig_ref[...] = sp + 0.001


def _head(af, wm1t, bm1, gm1, bgm1, wm2t, bm2, gm2, bgm2, wm3t, bm3):
    return pl.pallas_call(
        _head_kernel,
        out_shape=jax.ShapeDtypeStruct((BM, 1), jnp.float32),
    )(af, wm1t, bm1, gm1, bgm1, wm2t, bm2, gm2, bgm2, wm3t, bm3)


def kernel(xyz, features, W1, g1, b1, W2, g2, b2, W3, g3, b3,
           Wm1, bm1, gm1, bgm1, Wm2, bm2, gm2, bgm2, Wm3, bm3):
    perm = jax.random.permutation(jax.random.key(42), N)[:M]
    sampled = jnp.take(xyz, perm, axis=1)
    samp_t = jnp.transpose(sampled, (0, 2, 1))

    idx4 = _knn_topk(samp_t, xyz)
    idx = idx4.transpose(0, 1, 3, 2).reshape(B, M, K)

    pad = jnp.zeros((B, N, TABW - 67), jnp.float32)
    tab = jnp.concatenate([features, xyz, pad], axis=2).reshape(B * N, TABW)
    grows = _gather_rows(tab, idx.reshape(BT))
    grows3 = grows.reshape(BM, K, TABW)
    knn_xyz = grows3[:, :, 64:67]

    samp2 = sampled.reshape(BM, 3)
    x0, m0, s0 = _grouped(grows3, samp2)

    w1t, w2t, w3t = W1.T, W2.T, W3.T
    x1, m1, s1 = _conv_layer(x0, w1t, g1[None], b1[None], m0, s0,
                             68, 64, True)
    x2, m2, s2 = _conv_layer(x1, w2t, g2[None], b2[None], m1, s1,
                             64, 128, True)
    afm, af, kp = _attn(x2, w3t, g3[None], b3[None], m2, s2, knn_xyz)

    sig = _head(af, Wm1.T, bm1[None], gm1[None], bgm1[None],
                Wm2.T, bm2[None], gm2[None], bgm2[None], Wm3.T, bm3[None])

    keypoints = kp.reshape(B, M, 3)
    sigmas = sig.reshape(B, M)
    attentive_feature = af.reshape(B, M, 256).transpose(0, 2, 1)
    grouped_features = x0.reshape(B, M, K, 68).transpose(0, 3, 1, 2)
    attentive_feature_map = afm.reshape(B, M, K, 256).transpose(0, 3, 1, 2)
    return (keypoints, sigmas, attentive_feature, grouped_features,
            attentive_feature_map)


# speedup vs baseline: 1.0926x; 1.0043x over previous
import functools

import jax
import jax.numpy as jnp
from jax import lax
from jax.experimental import pallas as pl
from jax.experimental.pallas import tpu as pltpu
from jax.experimental.pallas import tpu_sc as plsc

B, N = 4, 16384
M, K = 1024, 32
BM = B * M
BT = BM * K
TABW = 128
MB = 256
Q = 256
P_CONV = float(BT)
EPS = 1e-5
HI = jax.lax.Precision.HIGHEST


MBL = 256
NC = 128
CL = N // NC
CSH = CL.bit_length() - 1
BIGI = 2 ** 30


def _knn_kernel(s_ref, st_ref, x_ref, idx_ref, vals_ref, cm_ref, cpos_ref,
                cm2_ref, cpos2_ref, buf_ref):
    b = pl.program_id(0)
    s = s_ref[0]
    st = st_ref[0]
    x = x_ref[0]
    s2 = jnp.sum(st * st, axis=0, keepdims=True)
    x2 = jnp.sum(x * x, axis=1, keepdims=True)
    dot = jax.lax.dot_general(x, s, (((1,), (1,)), ((), ())),
                              preferred_element_type=jnp.float32)
    d2 = (s2 + x2) - 2.0 * dot
    vals_ref[...] = jnp.reshape(d2, (NC, CL, MBL))

    p_iota3 = jax.lax.broadcasted_iota(jnp.int32, (NC, CL, MBL), 1)
    g_iota3 = (jax.lax.broadcasted_iota(jnp.int32, (NC, CL, MBL), 0) * CL
               + p_iota3)
    c_iota2 = jax.lax.broadcasted_iota(jnp.int32, (NC, MBL), 0)
    k_iota2 = jax.lax.broadcasted_iota(jnp.int32, (K, MBL), 0)
    INF = jnp.float32(jnp.inf)

    def refresh(lv, lg):
        v3 = vals_ref[...]
        done = (v3 < lv) | ((v3 == lv) & (g_iota3 <= lg))
        vm = jnp.where(done, INF, v3)
        cm = jnp.min(vm, axis=1)
        cpos = jnp.min(
            jnp.where(vm == cm[:, None, :], p_iota3, CL), axis=1)
        vm2 = jnp.where(p_iota3 == cpos[:, None, :], INF, vm)
        cm2 = jnp.min(vm2, axis=1)
        cpos2 = jnp.min(
            jnp.where(vm2 == cm2[:, None, :], p_iota3, CL), axis=1)
        cm_ref[...] = cm
        cpos_ref[...] = cpos.astype(jnp.int32)
        cm2_ref[...] = cm2
        cpos2_ref[...] = cpos2.astype(jnp.int32)

    def extract_step(_, carry):
        kq, guard, lv, lg = carry
        cm, cm2 = cm_ref[...], cm2_ref[...]
        mn = jnp.min(cm, axis=0, keepdims=True)
        gcand = jnp.where(cm == mn, c_iota2 * CL + cpos_ref[...], BIGI)
        gsel = jnp.min(gcand, axis=0, keepdims=True)
        valid = (mn < guard) & (kq < K)
        csel = jax.lax.shift_right_logical(gsel, CSH)
        hit = (c_iota2 == csel) & valid
        sec = hit & (cm2 == INF)
        anysec = jnp.max(jnp.where(sec, 1, 0), axis=0, keepdims=True) == 1
        cm_ref[...] = jnp.where(hit, jnp.where(sec, INF, cm2), cm)
        cpos_ref[...] = jnp.where(hit, cpos2_ref[...], cpos_ref[...])
        cm2_ref[...] = jnp.where(hit, INF, cm2)
        buf_ref[...] = jnp.where((k_iota2 == kq) & valid, gsel, buf_ref[...])
        lv = jnp.where(valid, mn, lv)
        lg = jnp.where(valid, gsel, lg)
        guard = jnp.where(anysec, jnp.minimum(guard, mn), guard)
        kq = jnp.where(valid, kq + 1, kq)
        return kq, guard, lv, lg

    def round_body(carry):
        r, kq, lv, lg = carry
        refresh(lv, lg)
        guard = jnp.full((1, MBL), INF)
        kq, _, lv, lg = jax.lax.fori_loop(
            0, K, extract_step, (kq, guard, lv, lg))
        return r + 1, kq, lv, lg

    def round_cond(carry):
        r, kq, lv, lg = carry
        return (r < K + 1) & (jnp.min(kq) < K)

    init = (jnp.int32(0), jnp.zeros((1, MBL), jnp.int32),
            jnp.full((1, MBL), -INF), jnp.full((1, MBL), -BIGI))
    jax.lax.while_loop(round_cond, round_body, init)
    idx_ref[0, 0] = buf_ref[...] + b * N


def _knn_topk(sampled, sampled_t, xyz):
    return pl.pallas_call(
        _knn_kernel,
        grid=(B, M // MBL),
        in_specs=[
            pl.BlockSpec((1, MBL, 3), lambda b, i: (b, i, 0)),
            pl.BlockSpec((1, 3, MBL), lambda b, i: (b, 0, i)),
            pl.BlockSpec((1, N, 3), lambda b, i: (b, 0, 0)),
        ],
        out_specs=pl.BlockSpec((1, 1, K, MBL), lambda b, i: (b, i, 0, 0)),
        out_shape=jax.ShapeDtypeStruct((B, M // MBL, K, MBL), jnp.int32),
        scratch_shapes=[pltpu.VMEM((NC, CL, MBL), jnp.float32),
                        pltpu.VMEM((NC, MBL), jnp.float32),
                        pltpu.VMEM((NC, MBL), jnp.int32),
                        pltpu.VMEM((NC, MBL), jnp.float32),
                        pltpu.VMEM((NC, MBL), jnp.int32),
                        pltpu.VMEM((K, MBL), jnp.int32)],
        compiler_params=pltpu.CompilerParams(
            vmem_limit_bytes=100 * 1024 * 1024),
    )(sampled, sampled_t, xyz)


def _gather_rows(tab, idx_flat):
    info = plsc.get_sparse_core_info()
    nw = info.num_cores * info.num_subcores
    b_per_w = BT // nw
    chunk = 128
    n_chunks = b_per_w // chunk

    @functools.partial(
        pl.kernel,
        mesh=plsc.VectorSubcoreMesh(core_axis_name="c", subcore_axis_name="s"),
        out_type=jax.ShapeDtypeStruct((BT, TABW), jnp.float32),
        scratch_types=[
            pltpu.VMEM((chunk,), jnp.int32),
            pltpu.VMEM((chunk, TABW), jnp.float32),
            pltpu.SemaphoreType.DMA,
        ],
    )
    def gather_k(tab_hbm, idx_hbm, out_hbm, idx_v, rows_v, sem):
        wid = lax.axis_index("s") * info.num_cores + lax.axis_index("c")
        base = wid * b_per_w

        def body(i, carry):
            off = base + i * chunk
            pltpu.sync_copy(idx_hbm.at[pl.ds(off, chunk)], idx_v)
            pltpu.async_copy(tab_hbm.at[idx_v], rows_v, sem).wait()
            pltpu.sync_copy(rows_v, out_hbm.at[pl.ds(off, chunk)])
            return carry

        lax.fori_loop(0, n_chunks, body, 0)

    return gather_k(tab, idx_flat)


def _grouped_kernel(g_ref, s_ref, x0_ref, m0_ref, s0_ref):
    @pl.when(pl.program_id(0) == 0)
    def _():
        m0_ref[...] = jnp.zeros_like(m0_ref)
        s0_ref[...] = jnp.zeros_like(s0_ref)

    g = g_ref[...]
    s = s_ref[...]
    feat = g[:, :, 0:64]
    xyzk = g[:, :, 64:67]
    rela = xyzk - s[:, None, :]
    dist = jnp.sqrt(jnp.sum(rela * rela, axis=2, keepdims=True))
    x0 = jnp.concatenate([rela, dist, feat], axis=2)
    x0_ref[...] = x0
    x2d = jnp.reshape(x0, (Q * K, 68))
    m0_ref[...] += jnp.sum(x2d, axis=0, keepdims=True)
    s0_ref[...] += jax.lax.dot_general(
        x2d, x2d, (((0,), (0,)), ((), ())), precision=HI,
        preferred_element_type=jnp.float32)


def _grouped(grows, samp):
    return pl.pallas_call(
        _grouped_kernel,
        grid=(BM // Q,),
        in_specs=[
            pl.BlockSpec((Q, K, TABW), lambda i: (i, 0, 0)),
            pl.BlockSpec((Q, 3), lambda i: (i, 0)),
        ],
        out_specs=[
            pl.BlockSpec((Q, K, 68), lambda i: (i, 0, 0)),
            pl.BlockSpec((1, 68), lambda i: (0, 0)),
            pl.BlockSpec((68, 68), lambda i: (0, 0)),
        ],
        out_shape=[
            jax.ShapeDtypeStruct((BM, K, 68), jnp.float32),
            jax.ShapeDtypeStruct((1, 68), jnp.float32),
            jax.ShapeDtypeStruct((68, 68), jnp.float32),
        ],
    )(grows, samp)


def _make_conv_kernel(cin, cout, with_moments):
    def body(x_ref, wt_ref, g_ref, b_ref, m_ref, s_ref, *rest):
        scale_ref, shift_ref = rest[-2:]
        if with_moments:
            y_ref, mo_ref, so_ref = rest[:-2]
        else:
            (y_ref,) = rest[:-2]

        @pl.when(pl.program_id(0) == 0)
        def _():
            wt = wt_ref[...]
            mu = jax.lax.dot_general(
                m_ref[...] / P_CONV, wt, (((1,), (0,)), ((), ())),
                precision=HI, preferred_element_type=jnp.float32)
            t = jax.lax.dot_general(
                s_ref[...] / P_CONV, wt, (((1,), (0,)), ((), ())),
                precision=HI, preferred_element_type=jnp.float32)
            e2 = jnp.sum(wt * t, axis=0, keepdims=True)
            var = e2 - mu * mu
            isd = jax.lax.rsqrt(var + EPS)
            scale_ref[...] = isd * g_ref[...]
            shift_ref[...] = b_ref[...] - mu * isd * g_ref[...]
            if with_moments:
                mo_ref[...] = jnp.zeros_like(mo_ref)
                so_ref[...] = jnp.zeros_like(so_ref)

        x = jnp.reshape(x_ref[...], (Q * K, cin))
        y = jax.lax.dot_general(x, wt_ref[...], (((1,), (0,)), ((), ())),
                                preferred_element_type=jnp.float32)
        y = jnp.maximum(y * scale_ref[...] + shift_ref[...], 0.0)
        y_ref[...] = jnp.reshape(y, (Q, K, cout))
        if with_moments:
            mo_ref[...] += jnp.sum(y, axis=0, keepdims=True)
            so_ref[...] += jax.lax.dot_general(
                y, y, (((0,), (0,)), ((), ())), precision=HI,
                preferred_element_type=jnp.float32)

    return body


def _conv_layer(x, wt, g, b, m, s, cin, cout, with_moments):
    kern = _make_conv_kernel(cin, cout, with_moments)
    out_specs = [pl.BlockSpec((Q, K, cout), lambda i: (i, 0, 0))]
    out_shape = [jax.ShapeDtypeStruct((BM, K, cout), jnp.float32)]
    if with_moments:
        out_specs += [pl.BlockSpec((1, cout), lambda i: (0, 0)),
                      pl.BlockSpec((cout, cout), lambda i: (0, 0))]
        out_shape += [jax.ShapeDtypeStruct((1, cout), jnp.float32),
                      jax.ShapeDtypeStruct((cout, cout), jnp.float32)]
    return pl.pallas_call(
        functools.partial(kern),
        grid=(BM // Q,),
        in_specs=[
            pl.BlockSpec((Q, K, cin), lambda i: (i, 0, 0)),
            pl.BlockSpec((cin, cout), lambda i: (0, 0)),
            pl.BlockSpec((1, cout), lambda i: (0, 0)),
            pl.BlockSpec((1, cout), lambda i: (0, 0)),
            pl.BlockSpec((1, cin), lambda i: (0, 0)),
            pl.BlockSpec((cin, cin), lambda i: (0, 0)),
        ],
        out_specs=out_specs,
        out_shape=out_shape,
        scratch_shapes=[pltpu.VMEM((1, cout), jnp.float32),
                        pltpu.VMEM((1, cout), jnp.float32)],
    )(x, wt, g, b, m, s)


def _attn_kernel(x_ref, wt_ref, g_ref, b_ref, m_ref, s_ref, kx_ref,
                 afm_ref, af_ref, kp_ref, scale_ref, shift_ref):
    @pl.when(pl.program_id(0) == 0)
    def _():
        wt = wt_ref[...]
        mu = jax.lax.dot_general(
            m_ref[...] / P_CONV, wt, (((1,), (0,)), ((), ())),
            precision=HI, preferred_element_type=jnp.float32)
        t = jax.lax.dot_general(
            s_ref[...] / P_CONV, wt, (((1,), (0,)), ((), ())),
            precision=HI, preferred_element_type=jnp.float32)
        e2 = jnp.sum(wt * t, axis=0, keepdims=True)
        var = e2 - mu * mu
        isd = jax.lax.rsqrt(var + EPS)
        scale_ref[...] = isd * g_ref[...]
        shift_ref[...] = b_ref[...] - mu * isd * g_ref[...]

    x = jnp.reshape(x_ref[...], (Q * K, 128))
    y = jax.lax.dot_general(x, wt_ref[...], (((1,), (0,)), ((), ())),
                            preferred_element_type=jnp.float32)
    emb = jnp.maximum(y * scale_ref[...] + shift_ref[...], 0.0)
    emb3 = jnp.reshape(emb, (Q, K, 256))
    x1max = jnp.max(emb3, axis=2)
    mx = jnp.max(x1max, axis=1, keepdims=True)
    e = jnp.exp(x1max - mx)
    aw = e / jnp.sum(e, axis=1, keepdims=True)
    kp_ref[...] = jnp.sum(aw[:, :, None] * kx_ref[...], axis=1)
    afm = emb3 * aw[:, :, None]
    afm_ref[...] = afm
    af_ref[...] = jnp.sum(afm, axis=1)


def _attn(x2, w3t, g3, b3, m2, s2, knn_xyz):
    return pl.pallas_call(
        _attn_kernel,
        grid=(BM // Q,),
        in_specs=[
            pl.BlockSpec((Q, K, 128), lambda i: (i, 0, 0)),
            pl.BlockSpec((128, 256), lambda i: (0, 0)),
            pl.BlockSpec((1, 256), lambda i: (0, 0)),
            pl.BlockSpec((1, 256), lambda i: (0, 0)),
            pl.BlockSpec((1, 128), lambda i: (0, 0)),
            pl.BlockSpec((128, 128), lambda i: (0, 0)),
            pl.BlockSpec((Q, K, 3), lambda i: (i, 0, 0)),
        ],
        out_specs=[
            pl.BlockSpec((Q, K, 256), lambda i: (i, 0, 0)),
            pl.BlockSpec((Q, 256), lambda i: (i, 0)),
            pl.BlockSpec((Q, 3), lambda i: (i, 0)),
        ],
        out_shape=[
            jax.ShapeDtypeStruct((BM, K, 256), jnp.float32),
            jax.ShapeDtypeStruct((BM, 256), jnp.float32),
            jax.ShapeDtypeStruct((BM, 3), jnp.float32),
        ],
        scratch_shapes=[pltpu.VMEM((1, 256), jnp.float32),
                        pltpu.VMEM((1, 256), jnp.float32)],
    )(x2, w3t, g3, b3, m2, s2, knn_xyz)


def _head_kernel(af_ref, w1t_ref, b1_ref, g1_ref, bg1_ref,
                 w2t_ref, b2_ref, g2_ref, bg2_ref, w3t_ref, b3_ref,
                 sig_ref):
    af = af_ref[...]

    def mlp_bn(x, wt, bb, g, b):
        y = jax.lax.dot_general(x, wt, (((1,), (0,)), ((), ())),
                                preferred_element_type=jnp.float32) + bb
        mu = jnp.mean(y, axis=0, keepdims=True)
        var = jnp.mean((y - mu) * (y - mu), axis=0, keepdims=True)
        return jnp.maximum((y - mu) * jax.lax.rsqrt(var + EPS) * g + b, 0.0)

    h = mlp_bn(af, w1t_ref[...], b1_ref[...], g1_ref[...], bg1_ref[...])
    h = mlp_bn(h, w2t_ref[...], b2_ref[...], g2_ref[...], bg2_ref[...])
    s = jax.lax.dot_general(h, w3t_ref[...], (((1,), (0,)), ((), ())),
                            preferred_element_type=jnp.float32) + b3_ref[...]
    sp = jnp.maximum(s, 0.0) + jnp.log(1.0 + jnp.exp(-jnp.abs(s)))
    sig_ref[...] = sp + 0.001


def _head(af, wm1t, bm1, gm1, bgm1, wm2t, bm2, gm2, bgm2, wm3t, bm3):
    return pl.pallas_call(
        _head_kernel,
        out_shape=jax.ShapeDtypeStruct((BM, 1), jnp.float32),
    )(af, wm1t, bm1, gm1, bgm1, wm2t, bm2, gm2, bgm2, wm3t, bm3)


def kernel(xyz, features, W1, g1, b1, W2, g2, b2, W3, g3, b3,
           Wm1, bm1, gm1, bgm1, Wm2, bm2, gm2, bgm2, Wm3, bm3):
    perm = jax.random.permutation(jax.random.key(42), N)[:M]
    sampled = jnp.take(xyz, perm, axis=1)
    samp_t = jnp.transpose(sampled, (0, 2, 1))

    idx4 = _knn_topk(sampled, samp_t, xyz)
    idx = idx4.transpose(0, 1, 3, 2).reshape(B, M, K)

    pad = jnp.zeros((B, N, TABW - 67), jnp.float32)
    tab = jnp.concatenate([features, xyz, pad], axis=2).reshape(B * N, TABW)
    grows = _gather_rows(tab, idx.reshape(BT))
    grows3 = grows.reshape(BM, K, TABW)
    knn_xyz = grows3[:, :, 64:67]

    samp2 = sampled.reshape(BM, 3)
    x0, m0, s0 = _grouped(grows3, samp2)

    w1t, w2t, w3t = W1.T, W2.T, W3.T
    x1, m1, s1 = _conv_layer(x0, w1t, g1[None], b1[None], m0, s0,
                             68, 64, True)
    x2, m2, s2 = _conv_layer(x1, w2t, g2[None], b2[None], m1, s1,
                             64, 128, True)
    afm, af, kp = _attn(x2, w3t, g3[None], b3[None], m2, s2, knn_xyz)

    sig = _head(af, Wm1.T, bm1[None], gm1[None], bgm1[None],
                Wm2.T, bm2[None], gm2[None], bgm2[None], Wm3.T, bm3[None])

    keypoints = kp.reshape(B, M, 3)
    sigmas = sig.reshape(B, M)
    attentive_feature = af.reshape(B, M, 256).transpose(0, 2, 1)
    grouped_features = x0.reshape(B, M, K, 68).transpose(0, 3, 1, 2)
    attentive_feature_map = afm.reshape(B, M, K, 256).transpose(0, 3, 1, 2)
    return (keypoints, sigmas, attentive_feature, grouped_features,
            attentive_feature_map)
